# baseline (speedup 1.0000x reference)
import jax
import jax.numpy as jnp
from jax import lax
from jax.experimental import pallas as pl
from jax.experimental.pallas import tpu as pltpu, tpu_sc as plsc

D = 256
N = 8192
NC = 2
NS = 16
NW = NC * NS
RPW = N // NW
L = 16
NQ = 4
CR = RPW // NQ

_MESH = plsc.VectorSubcoreMesh(
    core_axis_name="c", subcore_axis_name="s", num_cores=NC, num_subcores=NS)
_PARAMS = pltpu.CompilerParams(
    needs_layout_passes=False, skip_device_barrier=True)
_TCPARAMS = pltpu.CompilerParams(skip_device_barrier=True)


def _stage1(cb_hbm, z_hbm, out_d, out_i, out_r, cb_v, acc_v, z_v, res_v,
            resi_v, sems):
    c = lax.axis_index("c")
    s = lax.axis_index("s")
    wid = c * NS + s
    base_row = wid * RPW

    copies = [
        pltpu.async_copy(
            cb_hbm.at[pl.ds(base_row + q * CR, CR)],
            cb_v.at[pl.ds(q * CR, CR)], sems.at[q])
        for q in range(NQ)
    ]
    pltpu.sync_copy(z_hbm, z_v)
    z_vecs = [z_v[0, pl.ds(L * d, L)] for d in range(D // L)]

    iota = lax.broadcasted_iota(jnp.int32, (L,), 0)

    def row_body(r):
        for rr in range(2):
            row = r * 2 + rr
            a0 = jnp.zeros((L,), jnp.float32)
            a1 = jnp.zeros((L,), jnp.float32)
            for d in range(0, D // L, 2):
                t0 = cb_v[row, pl.ds(L * d, L)] - z_vecs[d]
                t1 = cb_v[row, pl.ds(L * (d + 1), L)] - z_vecs[d + 1]
                a0 = a0 + t0 * t0
                a1 = a1 + t1 * t1
            acc_v[row] = a0 + a1

    for q in range(NQ):
        copies[q].wait()
        plsc.parallel_loop(q * (CR // 2), (q + 1) * (CR // 2), unroll=2)(
            row_body)

    def tree_reduce(vecs):
        for k in range(4):
            bit = 1 << k
            mask = (iota & bit) != 0
            perm = iota ^ bit
            nxt = []
            for i in range(0, len(vecs), 2):
                a, b = vecs[i], vecs[i + 1]
                sel = jnp.where(mask, b, a)
                cross = jnp.where(
                    mask,
                    b.at[perm].get(mode="promise_in_bounds"),
                    a.at[perm].get(mode="promise_in_bounds"))
                nxt.append(sel + cross)
            vecs = nxt
        return vecs[0]

    def group_body(g, carry):
        best16, bidx16 = carry
        accs = [acc_v[g * L + r] for r in range(L)]
        w = tree_reduce(accs)
        gidx = (base_row + g * L) + iota
        m = w < best16
        best16 = jnp.where(m, w, best16)
        bidx16 = jnp.where(m, gidx, bidx16)
        return best16, bidx16

    best16, bidx16 = lax.fori_loop(
        0, RPW // L, group_body,
        (jnp.full((L,), jnp.inf, jnp.float32), jnp.zeros((L,), jnp.int32)))

    mn = jnp.min(best16)
    gi = jnp.min(jnp.where(best16 == mn, bidx16, jnp.int32(N)))
    res_v[...] = jnp.broadcast_to(mn, (L,))
    resi_v[...] = jnp.broadcast_to(gi, (L,))
    pltpu.sync_copy(res_v, out_d.at[pl.ds(wid * L, L)])
    pltpu.sync_copy(resi_v, out_i.at[pl.ds(wid * L, L)])
    pltpu.sync_copy(cb_v.at[gi - base_row], out_r.at[pl.ds(wid * D, D)])


def _merge(d_ref, i_ref, r_ref, row_ref, idx_ref, dist_ref):
    d = d_ref[...]
    i = i_ref[...]
    dmin = jnp.min(d)
    idx = jnp.min(jnp.where(d == dmin, i, jnp.int32(N)))
    idx_ref[...] = idx
    dist_ref[...] = jnp.sqrt(dmin)
    rows = r_ref[...].reshape(NW, D)
    am = idx // RPW
    sel = lax.broadcasted_iota(jnp.int32, (NW, 1), 0) == am
    row_ref[...] = jnp.sum(jnp.where(sel, rows, 0.0), axis=0)


@jax.jit
def kernel(z, codebook):
    s1 = pl.kernel(
        _stage1,
        out_type=(
            jax.ShapeDtypeStruct((NW * L,), jnp.float32),
            jax.ShapeDtypeStruct((NW * L,), jnp.int32),
            jax.ShapeDtypeStruct((NW * D,), jnp.float32),
        ),
        mesh=_MESH,
        compiler_params=_PARAMS,
        scratch_types=[
            pltpu.VMEM((RPW, D), jnp.float32),
            pltpu.VMEM((RPW, L), jnp.float32),
            pltpu.VMEM((1, D), jnp.float32),
            pltpu.VMEM((L,), jnp.float32),
            pltpu.VMEM((L,), jnp.int32),
            pltpu.SemaphoreType.DMA((NQ,)),
        ],
    )(codebook, z)
    out_d, out_i, out_r = s1

    row, idx, dist = pl.pallas_call(
        _merge,
        out_shape=(
            jax.ShapeDtypeStruct((D,), jnp.float32),
            jax.ShapeDtypeStruct((), jnp.int32),
            jax.ShapeDtypeStruct((), jnp.float32),
        ),
        compiler_params=_TCPARAMS,
        in_specs=[
            pl.BlockSpec(memory_space=pltpu.VMEM),
            pl.BlockSpec(memory_space=pltpu.VMEM),
            pl.BlockSpec(memory_space=pltpu.VMEM),
        ],
        out_specs=(
            pl.BlockSpec(memory_space=pltpu.VMEM),
            pl.BlockSpec(memory_space=pltpu.SMEM),
            pl.BlockSpec(memory_space=pltpu.SMEM),
        ),
    )(out_d, out_i, out_r)

    return row, idx, dist
